# baseline (speedup 1.0000x reference)
import jax
import jax.numpy as jnp
from jax import lax
from jax.experimental import pallas as pl
from jax.experimental.pallas import tpu as pltpu
from jax.experimental.pallas import tpu_sc as plsc

N_NODES = 100000
N_EDGES = 1600000
NPAD = 109824
NW = 32
EW = N_EDGES // 16
BLK = 2000
GROUPS = BLK // 16
NB = EW // BLK


def _make_seg(d, R, npasses, with_cnt):
  accrows = 3 * R + 128
  dump = 3 * R
  stripe = accrows // 16
  zfull, ztail = stripe // 128, stripe % 128
  chunks = R // 128
  kloop = (chunks + 15) // 16

  def body(*refs):
    if with_cnt:
      (h, srcr, dstr, etr, s_out, c_out, srcblk, dstblk, etblk, srcstage,
       idxstage, idxbuf, rows, zb, zb16, onesb, acc, cntacc, gsem, ssem,
       csem) = refs
    else:
      (h, srcr, dstr, etr, s_out, srcblk, dstblk, etblk, srcstage,
       idxstage, idxbuf, rows, zb, acc, gsem, ssem) = refs

    c = lax.axis_index("c")
    s = lax.axis_index("s")
    estart = s * EW

    def init_row(i, _):
      z16 = jnp.zeros((16,), jnp.float32)
      o16 = jnp.ones((16,), jnp.float32)
      for k in range(d // 16):
        zb[i, pl.ds(k * 16, 16)] = z16
      if with_cnt:
        zb16[i] = z16
        onesb[i] = o16
      return 0
    lax.fori_loop(0, 128, init_row, 0)

    for k in range(3):
      idxstage[pl.ds(BLK + k * 16, 16)] = jnp.full((16,), dump, jnp.int32)
      srcstage[pl.ds(BLK + k * 16, 16)] = jnp.zeros((16,), jnp.int32)

    def copy_chunk_to_idxbuf(j, src_off):
      for k in range(8):
        idxbuf[j, pl.ds(k * 16, 16)] = idxstage[pl.ds(src_off + k * 16, 16)]

    def pass_body(p, _):
      base = (2 * p + c) * R

      row0 = s * stripe
      for t in range(zfull):
        pltpu.sync_copy(zb, acc.at[pl.ds(row0 + t * 128, 128)])
        if with_cnt:
          pltpu.sync_copy(zb16, cntacc.at[pl.ds(row0 + t * 128, 128)])
      if ztail:
        pltpu.sync_copy(zb.at[pl.ds(0, ztail)],
                        acc.at[pl.ds(row0 + zfull * 128, ztail)])
        if with_cnt:
          pltpu.sync_copy(zb16.at[pl.ds(0, ztail)],
                          cntacc.at[pl.ds(row0 + zfull * 128, ztail)])
      plsc.subcore_barrier()

      def flush_half(hh):
        for j in range(8):
          copy_chunk_to_idxbuf(j, hh * 1024 + j * 128)
        g = [pltpu.async_copy(
            h.at[srcstage.at[pl.ds(hh * 1024 + j * 128, 128)]],
            rows.at[pl.ds(j * 128, 128)], gsem) for j in range(8)]
        for cp in g:
          cp.wait()
        sc = [pltpu.async_copy(rows.at[pl.ds(j * 128, 128)],
                               acc.at[idxbuf.at[j]], ssem, add=True)
              for j in range(8)]
        if with_cnt:
          cc = [pltpu.async_copy(onesb, cntacc.at[idxbuf.at[j]], csem,
                                 add=True) for j in range(8)]
        for cp in sc:
          cp.wait()
        if with_cnt:
          for cp in cc:
            cp.wait()

      def block_body(b, _):
        e0 = estart + b * BLK
        pltpu.sync_copy(srcr.at[pl.ds(e0, BLK)], srcblk)
        pltpu.sync_copy(dstr.at[pl.ds(e0, BLK)], dstblk)
        pltpu.sync_copy(etr.at[pl.ds(e0, BLK)], etblk)

        def g_body(g, _):
          o = g * 16
          d16 = dstblk[pl.ds(o, 16)]
          s16 = srcblk[pl.ds(o, 16)]
          t16 = etblk[pl.ds(o, 16)]
          u = d16 - base
          v = (base + R - 1) - d16
          neg = lax.shift_right_logical(jnp.bitwise_or(u, v), 31)
          mi = 1 - neg
          idxstage[pl.ds(o, 16)] = mi * (u + t16 * R) + neg * dump
          srcstage[pl.ds(o, 16)] = mi * s16
          return 0

        lax.fori_loop(0, GROUPS, g_body, 0)
        flush_half(0)
        flush_half(1)
        return 0

      lax.fori_loop(0, NB, block_body, 0)
      plsc.subcore_barrier()

      for t in range(3):
        for k in range(kloop):
          idx = k * 16 + s

          @pl.when(idx < chunks)
          def _():
            pltpu.sync_copy(acc.at[pl.ds(t * R + idx * 128, 128)],
                            s_out.at[t, pl.ds(base + idx * 128, 128)])
            if with_cnt:
              pltpu.sync_copy(cntacc.at[pl.ds(t * R + idx * 128, 128)],
                              c_out.at[t, pl.ds(base + idx * 128, 128)])
      plsc.subcore_barrier()
      return 0

    lax.fori_loop(0, npasses, pass_body, 0)

  out_type = [jax.ShapeDtypeStruct((3, NPAD, d), jnp.float32)]
  scratch = [
      pltpu.VMEM((BLK,), jnp.int32),
      pltpu.VMEM((BLK,), jnp.int32),
      pltpu.VMEM((BLK,), jnp.int32),
      pltpu.VMEM((2048,), jnp.int32),
      pltpu.VMEM((2048,), jnp.int32),
      pltpu.VMEM((8, 128), jnp.int32),
      pltpu.VMEM((1024, d), jnp.float32),
      pltpu.VMEM((128, d), jnp.float32),
  ]
  if with_cnt:
    out_type.append(jax.ShapeDtypeStruct((3, NPAD, 16), jnp.float32))
    scratch += [
        pltpu.VMEM((128, 16), jnp.float32),
        pltpu.VMEM((128, 16), jnp.float32),
    ]
  scratch.append(pltpu.VMEM_SHARED((accrows, d), jnp.float32))
  if with_cnt:
    scratch.append(pltpu.VMEM_SHARED((accrows, 16), jnp.float32))
  scratch.append(pltpu.SemaphoreType.DMA)
  scratch.append(pltpu.SemaphoreType.DMA)
  if with_cnt:
    scratch.append(pltpu.SemaphoreType.DMA)

  mesh = plsc.VectorSubcoreMesh(core_axis_name="c", subcore_axis_name="s")
  return pl.kernel(body, out_type=tuple(out_type), mesh=mesh,
                   scratch_types=tuple(scratch),
                   compiler_params=pltpu.CompilerParams(
                       use_tc_tiling_on_sc=False))


def _h0_body(x_ref, se_ref, ce_ref, wp_ref, bp_ref, o_ref):
  a2 = se_ref[...] @ wp_ref[0:8, :] + bp_ref[...]
  b2 = ce_ref[...] @ wp_ref[8:16, :]
  xv = x_ref[...]
  s_oh = (xv[:, 0:1] == lax.broadcasted_iota(jnp.int32, (1, 16), 1)
          ).astype(jnp.float32)
  c_oh = (xv[:, 1:2] == lax.broadcasted_iota(jnp.int32, (1, 8), 1)
          ).astype(jnp.float32)
  o_ref[...] = jnp.maximum(s_oh @ a2 + c_oh @ b2, 0.0)


def _layer_body(hp_ref, s_ref, c16_ref, w_ref, root_ref, b_ref, o_ref):
  out = hp_ref[...] @ root_ref[...] + b_ref[...]
  cnt = c16_ref[...]
  for t in range(3):
    inv = 1.0 / jnp.maximum(cnt[t, :, 0:1], 1.0)
    out = out + (s_ref[t] * inv) @ w_ref[t]
  o_ref[...] = jnp.maximum(out, 0.0)


def _cls_body(h_ref, w_ref, b_ref, o_ref):
  o_ref[...] = h_ref[...] @ w_ref[...] + b_ref[...]


def kernel(x, edge_index, edge_type, ptr, shape_emb, color_emb, W_pre,
           b_pre, W1, root1, b1, W2, root2, b2, W_cls, b_cls):
  src = edge_index[0]
  dst = edge_index[1]
  nb = N_NODES // 1000

  h0 = pl.pallas_call(
      _h0_body,
      grid=(nb,),
      in_specs=[
          pl.BlockSpec((1000, 2), lambda i: (i, 0)),
          pl.BlockSpec((16, 8), lambda i: (0, 0)),
          pl.BlockSpec((8, 8), lambda i: (0, 0)),
          pl.BlockSpec((16, 32), lambda i: (0, 0)),
          pl.BlockSpec((1, 32), lambda i: (0, 0)),
      ],
      out_specs=pl.BlockSpec((1000, 32), lambda i: (i, 0)),
      out_shape=jax.ShapeDtypeStruct((N_NODES, 32), jnp.float32),
  )(x, shape_emb, color_emb, W_pre, b_pre.reshape(1, 32))

  seg1 = _make_seg(32, 4992, 11, True)
  S1, cnt16 = seg1(h0, src, dst, edge_type)

  def _layer(hp, S, c16, W, root, b):
    d_in = hp.shape[1]
    return pl.pallas_call(
        _layer_body,
        grid=(nb,),
        in_specs=[
            pl.BlockSpec((1000, d_in), lambda i: (i, 0)),
            pl.BlockSpec((3, 1000, d_in), lambda i: (0, i, 0)),
            pl.BlockSpec((3, 1000, 16), lambda i: (0, i, 0)),
            pl.BlockSpec((3, d_in, 64), lambda i: (0, 0, 0)),
            pl.BlockSpec((d_in, 64), lambda i: (0, 0)),
            pl.BlockSpec((1, 64), lambda i: (0, 0)),
        ],
        out_specs=pl.BlockSpec((1000, 64), lambda i: (i, 0)),
        out_shape=jax.ShapeDtypeStruct((N_NODES, 64), jnp.float32),
    )(hp, S, c16, W, root, b.reshape(1, 64))

  h1 = _layer(h0, S1, cnt16, W1, root1, b1)

  seg2 = _make_seg(64, 3712, 14, False)
  (S2,) = seg2(h1, src, dst, edge_type)

  h2 = _layer(h1, S2, cnt16, W2, root2, b2)

  h_last = jnp.take(h2, ptr[1:] - 1, axis=0)
  return pl.pallas_call(
      _cls_body,
      in_specs=[
          pl.BlockSpec((256, 64), lambda: (0, 0)),
          pl.BlockSpec((64, 10), lambda: (0, 0)),
          pl.BlockSpec((1, 10), lambda: (0, 0)),
      ],
      out_specs=pl.BlockSpec((256, 10), lambda: (0, 0)),
      out_shape=jax.ShapeDtypeStruct((256, 10), jnp.float32),
  )(h_last, W_cls, b_cls.reshape(1, 10))
